# Initial kernel scaffold; baseline (speedup 1.0000x reference)
import functools

import jax
import jax.numpy as jnp
from jax import lax
from jax.experimental import pallas as pl
from jax.experimental.pallas import tpu as pltpu
from jax.experimental.pallas import tpu_sc as plsc

ROW_W = 128
NC, NS = 2, 16
NW = NC * NS
SLOTS = 8
DEPTH = 4


def _emb_kernel(idx_hbm, table_hbm, out_hbm, idx_v, rows_v, gsem, osem,
                *, rows_per_w):
    wid = lax.axis_index("s") * NC + lax.axis_index("c")
    base = wid * rows_per_w

    pltpu.sync_copy(idx_hbm.at[pl.ds(base, rows_per_w)], idx_v)

    for b in range(DEPTH):
        pltpu.async_copy(table_hbm.at[idx_v.at[b]], rows_v.at[b],
                         gsem.at[b])

    steps = rows_per_w // SLOTS

    def body(g, carry):
        for b in range(SLOTS):
            j = g * SLOTS + b
            bp = (b + DEPTH) % SLOTS
            jp = j + DEPTH

            @pl.when(jp < rows_per_w)
            def _():
                @pl.when(jp >= SLOTS)
                def _():
                    pltpu.make_async_copy(
                        rows_v.at[bp], out_hbm.at[0], osem.at[bp]).wait()

                pltpu.async_copy(table_hbm.at[idx_v.at[jp]], rows_v.at[bp],
                                 gsem.at[bp])

            pltpu.make_async_copy(
                table_hbm.at[idx_v.at[0]], rows_v.at[b], gsem.at[b]).wait()
            pltpu.async_copy(rows_v.at[b], out_hbm.at[base + j], osem.at[b])
        return carry

    lax.fori_loop(0, steps, body, 0)

    for b in range(SLOTS):
        pltpu.make_async_copy(rows_v.at[b], out_hbm.at[0], osem.at[b]).wait()


def kernel(input, table):
    batch, hist = input.shape
    n_vocab, d = table.shape
    total = batch * hist
    n_rows = total // ROW_W
    rows_per_w = n_rows // NW
    assert total == n_rows * ROW_W and n_rows == rows_per_w * NW
    assert rows_per_w % SLOTS == 0

    idx = input.reshape(n_rows, ROW_W).astype(jnp.int32)

    mesh = plsc.VectorSubcoreMesh(core_axis_name="c", subcore_axis_name="s")
    k = functools.partial(
        pl.kernel,
        mesh=mesh,
        out_type=jax.ShapeDtypeStruct((n_rows, ROW_W, d), jnp.float32),
        scratch_types=[
            pltpu.VMEM((rows_per_w, ROW_W), jnp.int32),
            pltpu.VMEM((SLOTS, ROW_W, d), jnp.float32),
            pltpu.SemaphoreType.DMA((SLOTS,)),
            pltpu.SemaphoreType.DMA((SLOTS,)),
        ],
    )(functools.partial(_emb_kernel, rows_per_w=rows_per_w))
    out = k(idx, table)
    return out.reshape(batch, hist, d)


# speedup vs baseline: 1.5004x; 1.5004x over previous
import functools

import jax
import jax.numpy as jnp
from jax import lax
from jax.experimental import pallas as pl
from jax.experimental.pallas import tpu as pltpu
from jax.experimental.pallas import tpu_sc as plsc

ROW_W = 128
NC, NS = 2, 16
NW = NC * NS
SLOTS = 8
DEPTH = 4


def _emb_kernel(idx_hbm, table_hbm, out_hbm, idx_v, rows_v, gsem, osem,
                *, rows_per_w):
    wid = lax.axis_index("s") * NC + lax.axis_index("c")
    base = wid * rows_per_w

    pltpu.sync_copy(idx_hbm.at[pl.ds(base, rows_per_w)], idx_v)

    for b in range(DEPTH):
        pltpu.async_copy(table_hbm.at[idx_v.at[b]], rows_v.at[b],
                         gsem.at[b])

    steps = rows_per_w // SLOTS

    def body(g, carry):
        for b in range(SLOTS):
            j = g * SLOTS + b
            bp = (b + DEPTH) % SLOTS
            jp = j + DEPTH

            @pl.when(jp < rows_per_w)
            def _():
                @pl.when(jp >= SLOTS)
                def _():
                    pltpu.make_async_copy(
                        rows_v.at[bp], out_hbm.at[0], osem.at[bp]).wait()

                pltpu.async_copy(table_hbm.at[idx_v.at[jp]], rows_v.at[bp],
                                 gsem.at[bp])

            pltpu.make_async_copy(
                table_hbm.at[idx_v.at[0]], rows_v.at[b], gsem.at[b]).wait()
            pltpu.async_copy(rows_v.at[b], out_hbm.at[base + j], osem.at[b])
        return carry

    lax.fori_loop(0, steps, body, 0)

    for b in range(SLOTS):
        pltpu.make_async_copy(rows_v.at[b], out_hbm.at[0], osem.at[b]).wait()


def kernel(input, table):
    batch, hist = input.shape
    n_vocab, d = table.shape
    total = batch * hist
    n_rows = total // ROW_W
    rows_per_w = n_rows // NW
    assert total == n_rows * ROW_W and n_rows == rows_per_w * NW
    assert rows_per_w % SLOTS == 0

    idx = input.reshape(n_rows, ROW_W).astype(jnp.int32)

    mesh = plsc.VectorSubcoreMesh(core_axis_name="c", subcore_axis_name="s")
    k = functools.partial(
        pl.kernel,
        mesh=mesh,
        compiler_params=pltpu.CompilerParams(use_tc_tiling_on_sc=False),
        out_type=jax.ShapeDtypeStruct((n_rows, ROW_W, d), jnp.float32),
        scratch_types=[
            pltpu.VMEM((rows_per_w, ROW_W), jnp.int32),
            pltpu.VMEM((SLOTS, ROW_W, d), jnp.float32),
            pltpu.SemaphoreType.DMA((SLOTS,)),
            pltpu.SemaphoreType.DMA((SLOTS,)),
        ],
    )(functools.partial(_emb_kernel, rows_per_w=rows_per_w))
    out = k(idx, table)
    return out.reshape(batch, hist, d)


# speedup vs baseline: 1.5017x; 1.0009x over previous
import functools

import jax
import jax.numpy as jnp
from jax import lax
from jax.experimental import pallas as pl
from jax.experimental.pallas import tpu as pltpu
from jax.experimental.pallas import tpu_sc as plsc

NC, NS = 2, 16
NW = NC * NS
GW = 640
NB = 4
PD = 2


def _emb_kernel(idx_hbm, table_hbm, out_hbm, idx_v, rows_v, gsem, osem,
                *, steps):
    wid = lax.axis_index("s") * NC + lax.axis_index("c")
    base = wid * steps

    pltpu.sync_copy(idx_hbm.at[pl.ds(base, steps)], idx_v)

    def gather_group(t, b):
        pltpu.async_copy(table_hbm.at[idx_v.at[t]], rows_v.at[b],
                         gsem.at[b])

    for b in range(PD):
        gather_group(b, b)

    def body(g, carry):
        for bs in range(NB):
            t = g * NB + bs
            tp = t + PD
            bp = (bs + PD) % NB

            @pl.when(tp < steps)
            def _():
                @pl.when(tp >= NB)
                def _():
                    pltpu.make_async_copy(
                        rows_v.at[bp], out_hbm.at[0], osem.at[bp]).wait()

                gather_group(tp, bp)

            pltpu.make_async_copy(
                table_hbm.at[idx_v.at[0]], rows_v.at[bs],
                gsem.at[bs]).wait()
            pltpu.async_copy(rows_v.at[bs], out_hbm.at[base + t],
                             osem.at[bs])
        return carry

    lax.fori_loop(0, steps // NB, body, 0)

    for b in range(NB):
        pltpu.make_async_copy(rows_v.at[b], out_hbm.at[0], osem.at[b]).wait()


def kernel(input, table):
    batch, hist = input.shape
    n_vocab, d = table.shape
    total = batch * hist
    n_grp = total // GW
    steps = n_grp // NW
    assert total == n_grp * GW and n_grp == steps * NW
    assert steps % NB == 0

    idx = input.reshape(n_grp, GW).astype(jnp.int32)

    mesh = plsc.VectorSubcoreMesh(core_axis_name="c", subcore_axis_name="s")
    k = functools.partial(
        pl.kernel,
        mesh=mesh,
        compiler_params=pltpu.CompilerParams(use_tc_tiling_on_sc=False),
        out_type=jax.ShapeDtypeStruct((n_grp, GW, d), jnp.float32),
        scratch_types=[
            pltpu.VMEM((steps, GW), jnp.int32),
            pltpu.VMEM((NB, GW, d), jnp.float32),
            pltpu.SemaphoreType.DMA((NB,)),
            pltpu.SemaphoreType.DMA((NB,)),
        ],
    )(functools.partial(_emb_kernel, steps=steps))
    out = k(idx, table)
    return out.reshape(batch, hist, d)
